# baseline (speedup 1.0000x reference)
import functools
import math

import jax
import jax.numpy as jnp
from jax import lax
from jax.experimental import pallas as pl
from jax.experimental.pallas import tpu as pltpu
from jax.experimental.pallas import tpu_sc as plsc


def _make_sc_scatter(EP, B, D):
    info = plsc.get_sparse_core_info()
    NC, NS = info.num_cores, info.num_subcores
    NW = NC * NS
    RPW = EP // NW
    mesh = plsc.VectorSubcoreMesh(core_axis_name="c", subcore_axis_name="s")

    @functools.partial(
        pl.kernel, mesh=mesh,
        out_type=jax.ShapeDtypeStruct((EP, D), jnp.float32),
        scratch_types=[
            pltpu.VMEM((B,), jnp.int32),
            pltpu.SMEM((RPW,), jnp.int32),
            pltpu.SemaphoreType.DMA,
            pltpu.SemaphoreType.DMA,
        ],
    )
    def sc_scatter(store_hbm, value_hbm, idx_hbm, out_hbm,
                   idx_v, patch_s, sem_cp, sem_row):
        wid = lax.axis_index("s") * NC + lax.axis_index("c")
        base = wid * RPW
        pltpu.sync_copy(idx_hbm, idx_v)
        cp = pltpu.make_async_copy(
            store_hbm.at[pl.ds(base, RPW)], out_hbm.at[pl.ds(base, RPW)],
            sem_cp)
        cp.start()

        def init_body(r, carry):
            patch_s[r] = -1
            return carry

        lax.fori_loop(0, RPW, init_body, 0)

        def scan_body(c, carry):
            iv = idx_v[pl.ds(c * 16, 16)]
            for l in range(16):
                r = iv[l] - base

                @pl.when((r >= 0) & (r < RPW))
                def _():
                    patch_s[r] = c * 16 + l
            return carry

        lax.fori_loop(0, B // 16, scan_body, 0)
        cp.wait()

        def fire_body(r, n):
            b = patch_s[r]

            @pl.when(b >= 0)
            def _():
                pltpu.make_async_copy(
                    value_hbm.at[b], out_hbm.at[base + r], sem_row).start()
            return n + (b >= 0).astype(jnp.int32)

        n_fired = lax.fori_loop(0, RPW, fire_body, jnp.int32(0))

        def drain_body(i, carry):
            pltpu.make_async_copy(
                value_hbm.at[0], out_hbm.at[base], sem_row).wait()
            return carry

        lax.fori_loop(0, n_fired, drain_body, 0)

    return sc_scatter


def _scatter_body(idx_ref, value_ref, store_ref, out_ref):
    c = pl.program_id(0)
    R = out_ref.shape[0]
    B = value_ref.shape[0]
    rows = jax.lax.broadcasted_iota(jnp.int32, (R, B), 0) + c * R
    biota = jax.lax.broadcasted_iota(jnp.int32, (R, B), 1)
    idx = idx_ref[0, :]
    hit = rows == idx[None, :]
    wb = jnp.max(jnp.where(hit, biota, -1), axis=1, keepdims=True)
    P = (biota == wb).astype(jnp.float32)
    corr = jax.lax.dot_general(
        P, value_ref[...], (((1,), (0,)), ((), ())),
        preferred_element_type=jnp.float32)
    out_ref[...] = jnp.where(wb >= 0, corr, store_ref[...])


def _sem_body(q_ref, k_ref, v_ref, out_ref, m_s, l_s, acc_s, *, scale, nk):
    j = pl.program_id(0)

    @pl.when(j == 0)
    def _():
        m_s[...] = jnp.full_like(m_s[...], -jnp.inf)
        l_s[...] = jnp.zeros_like(l_s[...])
        acc_s[...] = jnp.zeros_like(acc_s[...])

    qb = q_ref[...].astype(jnp.bfloat16)
    kb = k_ref[...].astype(jnp.bfloat16)
    s = jax.lax.dot_general(
        qb, kb, (((1,), (1,)), ((), ())),
        preferred_element_type=jnp.float32) * scale
    m_old = m_s[...]
    m_new = jnp.maximum(m_old, jnp.max(s, axis=1, keepdims=True))
    alpha = jnp.exp(m_old - m_new)
    p = jnp.exp(s - m_new[:, :1])
    l_s[...] = l_s[...] * alpha + jnp.sum(p, axis=1, keepdims=True)
    m_s[...] = m_new
    pv = jax.lax.dot_general(
        p.astype(jnp.bfloat16), v_ref[...].astype(jnp.bfloat16),
        (((1,), (0,)), ((), ())), preferred_element_type=jnp.float32)
    acc_s[...] = acc_s[...] * alpha[:, :1] + pv

    @pl.when(j == nk - 1)
    def _():
        out_ref[...] = 0.425 * acc_s[...] / l_s[...][:, :1]


def _ep_body(q_ref, ep_ref, partial_ref, out_ref,
             m1, l1, acc1, m2, l2, acc2, *, scale, beta, nk):
    j = pl.program_id(0)

    @pl.when(j == 0)
    def _():
        for m_s, l_s, acc_s in ((m1, l1, acc1), (m2, l2, acc2)):
            m_s[...] = jnp.full_like(m_s[...], -jnp.inf)
            l_s[...] = jnp.zeros_like(l_s[...])
            acc_s[...] = jnp.zeros_like(acc_s[...])

    ep = ep_ref[...]
    s0 = jax.lax.dot_general(
        q_ref[...], ep, (((1,), (1,)), ((), ())),
        preferred_element_type=jnp.float32)
    for m_s, l_s, acc_s, t in ((m1, l1, acc1, scale), (m2, l2, acc2, beta)):
        s = s0 * t
        m_old = m_s[...]
        m_new = jnp.maximum(m_old, jnp.max(s, axis=1, keepdims=True))
        alpha = jnp.exp(m_old - m_new)
        p = jnp.exp(s - m_new[:, :1])
        l_s[...] = l_s[...] * alpha + jnp.sum(p, axis=1, keepdims=True)
        m_s[...] = m_new
        pv = jax.lax.dot_general(
            p.astype(jnp.bfloat16), ep.astype(jnp.bfloat16),
            (((1,), (0,)), ((), ())),
            preferred_element_type=jnp.float32)
        acc_s[...] = acc_s[...] * alpha[:, :1] + pv

    @pl.when(j == nk - 1)
    def _():
        out_ref[...] = (partial_ref[...]
                        + 0.425 * acc1[...] / l1[...][:, :1]
                        + 0.15 * acc2[...] / l2[...][:, :1])


def kernel(query, value, episodic_store, semantic_keys, semantic_values,
           write_idx):
    B, D = query.shape
    EP = episodic_store.shape[0]
    SEM = semantic_keys.shape[0]
    scale = 1.0 / math.sqrt(D)
    beta = 2.0

    BQ = 1024
    BK_SEM = 1024
    BK_EP = 1024
    nk_sem = SEM // BK_SEM
    nk_ep = EP // BK_EP

    idx2d = write_idx.astype(jnp.int32).reshape(1, B)

    partial = pl.pallas_call(
        functools.partial(_sem_body, scale=scale, nk=nk_sem),
        grid=(nk_sem,),
        in_specs=[
            pl.BlockSpec((BQ, D), lambda j: (0, 0)),
            pl.BlockSpec((BK_SEM, D), lambda j: (j, 0)),
            pl.BlockSpec((BK_SEM, D), lambda j: (j, 0)),
        ],
        out_specs=pl.BlockSpec((BQ, D), lambda j: (0, 0)),
        out_shape=jax.ShapeDtypeStruct((B, D), jnp.float32),
        scratch_shapes=[
            pltpu.VMEM((BQ, 128), jnp.float32),
            pltpu.VMEM((BQ, 128), jnp.float32),
            pltpu.VMEM((BQ, D), jnp.float32),
        ],
        compiler_params=pltpu.CompilerParams(
            dimension_semantics=("arbitrary",)),
    )(query, semantic_keys, semantic_values)

    ep = _make_sc_scatter(EP, B, D)(
        episodic_store, value, write_idx.astype(jnp.int32))

    out = pl.pallas_call(
        functools.partial(_ep_body, scale=scale, beta=beta, nk=nk_ep),
        grid=(nk_ep,),
        in_specs=[
            pl.BlockSpec((BQ, D), lambda j: (0, 0)),
            pl.BlockSpec((BK_EP, D), lambda j: (j, 0)),
            pl.BlockSpec((BQ, D), lambda j: (0, 0)),
        ],
        out_specs=pl.BlockSpec((BQ, D), lambda j: (0, 0)),
        out_shape=jax.ShapeDtypeStruct((B, D), jnp.float32),
        scratch_shapes=[
            pltpu.VMEM((BQ, 128), jnp.float32),
            pltpu.VMEM((BQ, 128), jnp.float32),
            pltpu.VMEM((BQ, D), jnp.float32),
            pltpu.VMEM((BQ, 128), jnp.float32),
            pltpu.VMEM((BQ, 128), jnp.float32),
            pltpu.VMEM((BQ, D), jnp.float32),
        ],
        compiler_params=pltpu.CompilerParams(
            dimension_semantics=("arbitrary",)),
    )(query, ep, partial)

    return out


# speedup vs baseline: 3.4153x; 3.4153x over previous
import functools
import math

import jax
import jax.numpy as jnp
from jax import lax
from jax.experimental import pallas as pl
from jax.experimental.pallas import tpu as pltpu
from jax.experimental.pallas import tpu_sc as plsc


def _patch_body(idx_ref, patch_ref, *, B):
    c = pl.program_id(0)
    R = patch_ref.shape[0]
    rows = jax.lax.broadcasted_iota(jnp.int32, (R, B), 0) + c * R
    biota = jax.lax.broadcasted_iota(jnp.int32, (R, B), 1)
    m = rows == idx_ref[0, :][None, :]
    patch_ref[...] = jnp.max(jnp.where(m, biota, -1), axis=1, keepdims=True)


def _make_sc_gather(EP, B, D):
    info = plsc.get_sparse_core_info()
    NC, NS = info.num_cores, info.num_subcores
    NW = NC * NS
    RPW = EP // NW
    mesh = plsc.VectorSubcoreMesh(core_axis_name="c", subcore_axis_name="s")

    @functools.partial(
        pl.kernel, mesh=mesh,
        out_type=jax.ShapeDtypeStruct((EP, D), jnp.float32),
        scratch_types=[
            pltpu.VMEM((RPW,), jnp.int32),
            pltpu.SemaphoreType.DMA,
        ],
    )
    def sc_gather(value_hbm, patch_hbm, out_hbm, patch_v, sem_row):
        wid = lax.axis_index("s") * NC + lax.axis_index("c")
        base = wid * RPW
        pltpu.sync_copy(patch_hbm.at[pl.ds(base, RPW)], patch_v)

        def fire_chunk(c, n):
            pv = patch_v[pl.ds(c * 16, 16)]
            for l in range(16):
                b = pv[l]

                @pl.when(b >= 0)
                def _():
                    pltpu.make_async_copy(
                        value_hbm.at[b], out_hbm.at[base + c * 16 + l],
                        sem_row).start()
                n = n + (b >= 0).astype(jnp.int32)
            return n

        n_fired = lax.fori_loop(0, RPW // 16, fire_chunk, jnp.int32(0))

        def drain_body(i, carry):
            pltpu.make_async_copy(
                value_hbm.at[0], out_hbm.at[base], sem_row).wait()
            return carry

        lax.fori_loop(0, n_fired, drain_body, 0)

    return sc_gather


def _sem_body(q_ref, k_ref, v_ref, out_ref, m_s, l_s, acc_s, *, scale, nk):
    j = pl.program_id(0)

    @pl.when(j == 0)
    def _():
        m_s[...] = jnp.full_like(m_s[...], -jnp.inf)
        l_s[...] = jnp.zeros_like(l_s[...])
        acc_s[...] = jnp.zeros_like(acc_s[...])

    qb = q_ref[...].astype(jnp.bfloat16)
    kb = k_ref[...].astype(jnp.bfloat16)
    s = jax.lax.dot_general(
        qb, kb, (((1,), (1,)), ((), ())),
        preferred_element_type=jnp.float32) * scale
    m_old = m_s[...]
    m_new = jnp.maximum(m_old, jnp.max(s, axis=1, keepdims=True))
    alpha = jnp.exp(m_old - m_new)
    p = jnp.exp(s - m_new[:, :1])
    l_s[...] = l_s[...] * alpha + jnp.sum(p, axis=1, keepdims=True)
    m_s[...] = m_new
    pv = jax.lax.dot_general(
        p.astype(jnp.bfloat16), v_ref[...].astype(jnp.bfloat16),
        (((1,), (0,)), ((), ())), preferred_element_type=jnp.float32)
    acc_s[...] = acc_s[...] * alpha[:, :1] + pv

    @pl.when(j == nk - 1)
    def _():
        out_ref[...] = 0.425 * acc_s[...] / l_s[...][:, :1]


def _ep_body(q_ref, store_ref, patched_ref, pm_ref, partial_ref, out_ref,
             m1, l1, acc1, m2, l2, acc2, *, scale, beta, nk):
    j = pl.program_id(0)

    @pl.when(j == 0)
    def _():
        for m_s, l_s, acc_s in ((m1, l1, acc1), (m2, l2, acc2)):
            m_s[...] = jnp.full_like(m_s[...], -jnp.inf)
            l_s[...] = jnp.zeros_like(l_s[...])
            acc_s[...] = jnp.zeros_like(acc_s[...])

    pm = pm_ref[...]
    ep = jnp.where(pm >= 0, patched_ref[...], store_ref[...])
    s0 = jax.lax.dot_general(
        q_ref[...], ep, (((1,), (1,)), ((), ())),
        preferred_element_type=jnp.float32)
    epb = ep.astype(jnp.bfloat16)
    for m_s, l_s, acc_s, t in ((m1, l1, acc1, scale), (m2, l2, acc2, beta)):
        s = s0 * t
        m_old = m_s[...]
        m_new = jnp.maximum(m_old, jnp.max(s, axis=1, keepdims=True))
        alpha = jnp.exp(m_old - m_new)
        p = jnp.exp(s - m_new[:, :1])
        l_s[...] = l_s[...] * alpha + jnp.sum(p, axis=1, keepdims=True)
        m_s[...] = m_new
        pv = jax.lax.dot_general(
            p.astype(jnp.bfloat16), epb, (((1,), (0,)), ((), ())),
            preferred_element_type=jnp.float32)
        acc_s[...] = acc_s[...] * alpha[:, :1] + pv

    @pl.when(j == nk - 1)
    def _():
        out_ref[...] = (partial_ref[...]
                        + 0.425 * acc1[...] / l1[...][:, :1]
                        + 0.15 * acc2[...] / l2[...][:, :1])


def kernel(query, value, episodic_store, semantic_keys, semantic_values,
           write_idx):
    B, D = query.shape
    EP = episodic_store.shape[0]
    SEM = semantic_keys.shape[0]
    scale = 1.0 / math.sqrt(D)
    beta = 2.0

    BQ = 1024
    BK_SEM = 1024
    BK_EP = 1024
    nk_sem = SEM // BK_SEM
    nk_ep = EP // BK_EP

    idx2d = write_idx.astype(jnp.int32).reshape(1, B)

    RCH = 512
    patch = pl.pallas_call(
        functools.partial(_patch_body, B=B),
        grid=(EP // RCH,),
        in_specs=[pl.BlockSpec((1, B), lambda c: (0, 0))],
        out_specs=pl.BlockSpec((RCH, 1), lambda c: (c, 0)),
        out_shape=jax.ShapeDtypeStruct((EP, 1), jnp.int32),
    )(idx2d)
    patch_flat = patch.reshape(EP)

    partial = pl.pallas_call(
        functools.partial(_sem_body, scale=scale, nk=nk_sem),
        grid=(nk_sem,),
        in_specs=[
            pl.BlockSpec((BQ, D), lambda j: (0, 0)),
            pl.BlockSpec((BK_SEM, D), lambda j: (j, 0)),
            pl.BlockSpec((BK_SEM, D), lambda j: (j, 0)),
        ],
        out_specs=pl.BlockSpec((BQ, D), lambda j: (0, 0)),
        out_shape=jax.ShapeDtypeStruct((B, D), jnp.float32),
        scratch_shapes=[
            pltpu.VMEM((BQ, 128), jnp.float32),
            pltpu.VMEM((BQ, 128), jnp.float32),
            pltpu.VMEM((BQ, D), jnp.float32),
        ],
        compiler_params=pltpu.CompilerParams(
            dimension_semantics=("arbitrary",)),
    )(query, semantic_keys, semantic_values)

    patched = _make_sc_gather(EP, B, D)(value, patch_flat)

    out = pl.pallas_call(
        functools.partial(_ep_body, scale=scale, beta=beta, nk=nk_ep),
        grid=(nk_ep,),
        in_specs=[
            pl.BlockSpec((BQ, D), lambda j: (0, 0)),
            pl.BlockSpec((BK_EP, D), lambda j: (j, 0)),
            pl.BlockSpec((BK_EP, D), lambda j: (j, 0)),
            pl.BlockSpec((BK_EP, 1), lambda j: (j, 0)),
            pl.BlockSpec((BQ, D), lambda j: (0, 0)),
        ],
        out_specs=pl.BlockSpec((BQ, D), lambda j: (0, 0)),
        out_shape=jax.ShapeDtypeStruct((B, D), jnp.float32),
        scratch_shapes=[
            pltpu.VMEM((BQ, 128), jnp.float32),
            pltpu.VMEM((BQ, 128), jnp.float32),
            pltpu.VMEM((BQ, D), jnp.float32),
            pltpu.VMEM((BQ, 128), jnp.float32),
            pltpu.VMEM((BQ, 128), jnp.float32),
            pltpu.VMEM((BQ, D), jnp.float32),
        ],
        compiler_params=pltpu.CompilerParams(
            dimension_semantics=("arbitrary",)),
    )(query, episodic_store, patched, patch, partial)

    return out
